# baseline (speedup 1.0000x reference)
import dataclasses
import functools

import jax
import jax.numpy as jnp
from jax import lax
from jax.experimental import pallas as pl
from jax.experimental.pallas import tpu as pltpu
from jax.experimental.pallas import tpu_sc as plsc

D = 1024
LANES = 16
NC, NS = 2, 16
NW = NC * NS
CHUNK = 16
NBUF = 2


def _sc_gather_add(x3d, pids, table):
    nb, lp1, _ = x3d.shape
    w_per_b = NW // nb
    b_per_w = lp1 // w_per_b
    n_chunks = b_per_w // CHUNK
    mesh = plsc.VectorSubcoreMesh(core_axis_name="c", subcore_axis_name="s")

    cp = pltpu.CompilerParams()
    if "needs_layout_passes" in pltpu.CompilerParams.__dataclass_fields__:
        cp = dataclasses.replace(cp, needs_layout_passes=False)

    buf = lambda: pltpu.VMEM((CHUNK, D), jnp.float32)
    @functools.partial(
        pl.kernel,
        mesh=mesh,
        compiler_params=cp,
        out_type=jax.ShapeDtypeStruct((nb, lp1, D), jnp.float32),
        scratch_types=[
            pltpu.VMEM((b_per_w,), jnp.int32),
            pltpu.VMEM((nb, lp1 - 1), jnp.int32),
            buf(), buf(),
            buf(), buf(),
            buf(), buf(),
            pltpu.SemaphoreType.DMA, pltpu.SemaphoreType.DMA,
            pltpu.SemaphoreType.DMA, pltpu.SemaphoreType.DMA,
            pltpu.SemaphoreType.DMA, pltpu.SemaphoreType.DMA,
        ],
    )
    def k(table_hbm, pids_hbm, x_hbm, out_hbm, idx_v, tmp_v,
          pe0, pe1, xv0, xv1, ov0, ov1, g0, g1, xs0, xs1, os0, os1):
        pe_v, x_v, o_v = (pe0, pe1), (xv0, xv1), (ov0, ov1)
        gsem, xsem, osem = (g0, g1), (xs0, xs1), (os0, os1)

        wid = lax.axis_index("s") * NC + lax.axis_index("c")
        b_idx = wid // w_per_b
        col0 = (wid % w_per_b) * b_per_w

        pltpu.sync_copy(pids_hbm, tmp_v)

        iota = lax.iota(jnp.int32, LANES)
        row = jnp.full((LANES,), b_idx, jnp.int32)

        @pl.loop(0, b_per_w // LANES)
        def _grp(j):
            gcol = col0 + j * LANES + iota
            vals = plsc.load_gather(tmp_v, [row, jnp.maximum(gcol - 1, 0)])
            idx_v[pl.ds(j * LANES, LANES)] = jnp.where(gcol == 0, 0, vals)

        def start_fetch(c, b):
            pltpu.async_copy(
                table_hbm.at[idx_v.at[pl.ds(c * CHUNK, CHUNK)]], pe_v[b], gsem[b]
            )
            pltpu.async_copy(
                x_hbm.at[b_idx, pl.ds(col0 + c * CHUNK, CHUNK)], x_v[b], xsem[b]
            )

        def wait_fetch(c, b):
            pltpu.make_async_copy(
                table_hbm.at[idx_v.at[pl.ds(c * CHUNK, CHUNK)]], pe_v[b], gsem[b]
            ).wait()
            pltpu.make_async_copy(
                x_hbm.at[b_idx, pl.ds(col0 + c * CHUNK, CHUNK)], x_v[b], xsem[b]
            ).wait()

        def out_copy(c, b):
            return pltpu.make_async_copy(
                o_v[b], out_hbm.at[b_idx, pl.ds(col0 + c * CHUNK, CHUNK)], osem[b]
            )

        for b in range(NBUF):
            start_fetch(b, b)

        @pl.loop(0, n_chunks, step=NBUF)
        def _pair(c0):
            for b in range(NBUF):
                c = c0 + b
                wait_fetch(c, b)

                @pl.when(c0 > 0)
                def _():
                    out_copy(c - NBUF, b).wait()

                @pl.loop(0, CHUNK)
                def _row(r):
                    for j in range(D // LANES):
                        sl = (r, pl.ds(j * LANES, LANES))
                        o_v[b][sl] = pe_v[b][sl] + x_v[b][sl]

                out_copy(c, b).start()

                @pl.when(c + NBUF < n_chunks)
                def _():
                    start_fetch(c + NBUF, b)

        for b in range(NBUF):
            out_copy(n_chunks - NBUF + b, b).wait()

    return k(table, pids, x3d)


def kernel(x, position_ids, pe_table):
    return _sc_gather_add(x, position_ids.astype(jnp.int32), pe_table)
